# baseline (speedup 1.0000x reference)
import functools

import jax
import jax.numpy as jnp
from jax import lax
from jax.experimental import pallas as pl
from jax.experimental.pallas import tpu as pltpu
from jax.experimental.pallas import tpu_sc as plsc

N_NODES = 10000
N_EDGES = 30000
NUM_GRAPHS = 512
DIM_HIDDEN = 32
DIM_EDGE_EMBED = 128
DIM_OUTPUT = 64

NPAD = 10240
EPAD = 32768
DPADH = 128
EBLK = 2048
NBLK = 1024
NC, NS = 2, 16
NW = NC * NS


@functools.lru_cache(maxsize=None)
def _make_gather(d, erows):
    rows_per_w = erows // NW
    k = rows_per_w // 128
    mesh = plsc.VectorSubcoreMesh(
        core_axis_name="c", subcore_axis_name="s", num_cores=NC, num_subcores=NS
    )

    @functools.partial(
        pl.kernel,
        out_type=jax.ShapeDtypeStruct((erows, d), jnp.float32),
        mesh=mesh,
        scratch_types=[
            pltpu.VMEM((k, 128), jnp.int32),
            pltpu.VMEM((128, d), jnp.float32),
            pltpu.VMEM((128, d), jnp.float32),
            pltpu.VMEM((128, d), jnp.float32),
            pltpu.SemaphoreType.DMA,
            pltpu.SemaphoreType.DMA,
            pltpu.SemaphoreType.DMA,
        ],
    )
    def gather(table_hbm, idx_hbm, out_hbm, idx_v,
               rows_a, rows_b, rows_c, sem_a, sem_b, sem_c):
        cid = lax.axis_index("c")
        sid = lax.axis_index("s")
        wid = sid * NC + cid
        pltpu.sync_copy(idx_hbm.at[pl.ds(wid * k, k)], idx_v)

        bufs = [(rows_a, sem_a), (rows_b, sem_b), (rows_c, sem_c)]
        depth = 3
        copies = [None] * k
        for j in range(min(depth, k)):
            buf, sem = bufs[j % depth]
            copies[j] = pltpu.async_copy(table_hbm.at[idx_v.at[j]], buf, sem)
        for j in range(k):
            copies[j].wait()
            pltpu.sync_copy(
                bufs[j % depth][0],
                out_hbm.at[pl.ds(wid * rows_per_w + j * 128, 128)],
            )
            if j + depth < k:
                buf, sem = bufs[(j + depth) % depth]
                copies[j + depth] = pltpu.async_copy(
                    table_hbm.at[idx_v.at[j + depth]], buf, sem
                )

    return gather


@functools.lru_cache(maxsize=None)
def _make_scatter(d, erows, trows):
    nch = erows // 128
    nmine = nch // NS
    half = trows // 2
    tloc = half + 128
    per_tile = half // NS
    mesh = plsc.VectorSubcoreMesh(
        core_axis_name="c", subcore_axis_name="s", num_cores=NC, num_subcores=NS
    )

    @functools.partial(
        pl.kernel,
        out_type=jax.ShapeDtypeStruct((trows, d), jnp.float32),
        mesh=mesh,
        scratch_types=[
            pltpu.VMEM((nmine, 128), jnp.int32),
            pltpu.VMEM((128, d), jnp.float32),
            pltpu.VMEM((128, d), jnp.float32),
            pltpu.VMEM((per_tile, d), jnp.float32),
            pltpu.VMEM_SHARED((tloc, d), jnp.float32),
            pltpu.SemaphoreType.DMA,
            pltpu.SemaphoreType.DMA,
        ],
    )
    def scatter(msg_hbm, idx_hbm, init_hbm, out_hbm, idx_v, rows_a, rows_b,
                stage_v, acc_sh, sem_a, sem_b):
        cid = lax.axis_index("c")
        sid = lax.axis_index("s")
        lo = cid * half
        base = sid * nmine
        bufs = [(rows_a, sem_a), (rows_b, sem_b)]
        copies = [None] * nmine
        copies[0] = pltpu.async_copy(
            msg_hbm.at[pl.ds(base * 128, 128)], rows_a, sem_a
        )
        pltpu.sync_copy(idx_hbm.at[pl.ds(base, nmine)], idx_v)
        pltpu.sync_copy(init_hbm.at[pl.ds(lo + sid * per_tile, per_tile)], stage_v)
        pltpu.sync_copy(stage_v, acc_sh.at[pl.ds(sid * per_tile, per_tile)])
        for t in range(nmine):
            for q in range(8):
                v = idx_v[t, pl.ds(q * 16, 16)] - lo
                ok = (v >= 0) & (v < half)
                idx_v[t, pl.ds(q * 16, 16)] = jnp.where(ok, v, half)
        plsc.subcore_barrier()

        for t in range(nmine):
            if t + 1 < nmine:
                buf, sem = bufs[(t + 1) % 2]
                copies[t + 1] = pltpu.async_copy(
                    msg_hbm.at[pl.ds((base + t + 1) * 128, 128)], buf, sem
                )
            copies[t].wait()
            pltpu.sync_copy(bufs[t % 2][0], acc_sh.at[idx_v.at[t]], add=True)
        plsc.subcore_barrier()
        pltpu.sync_copy(acc_sh.at[pl.ds(sid * per_tile, per_tile)], stage_v)
        pltpu.sync_copy(
            stage_v, out_hbm.at[pl.ds(lo + sid * per_tile, per_tile)]
        )

    return scatter


def _msg_body(ea_ref, xj_ref, w1_ref, b1_ref, w2_ref, b2r_ref, out_ref):
    e = jnp.maximum(
        jnp.dot(ea_ref[...], w1_ref[...], preferred_element_type=jnp.float32)
        + b1_ref[...],
        0.0,
    )
    t2 = jnp.dot(xj_ref[...], w2_ref[...], preferred_element_type=jnp.float32)
    u = (t2.reshape(EBLK, DIM_HIDDEN, 128) * e[:, None, :]).sum(axis=-1)
    u128 = jnp.concatenate(
        [u, jnp.zeros((EBLK, DPADH - DIM_HIDDEN), jnp.float32)], axis=-1
    )
    out_ref[...] = u128 + jnp.dot(
        xj_ref[...], b2r_ref[...], preferred_element_type=jnp.float32
    )


def _msg_call(dpad, erows, ea, xj, w1p, b1r, w2p, b2r):
    grid = (erows // EBLK,)
    return pl.pallas_call(
        _msg_body,
        grid=grid,
        in_specs=[
            pl.BlockSpec((EBLK, 16), lambda i: (i, 0)),
            pl.BlockSpec((EBLK, dpad), lambda i: (i, 0)),
            pl.BlockSpec((16, 128), lambda i: (0, 0)),
            pl.BlockSpec((1, 128), lambda i: (0, 0)),
            pl.BlockSpec((dpad, DIM_HIDDEN * 128), lambda i: (0, 0)),
            pl.BlockSpec((dpad, DPADH), lambda i: (0, 0)),
        ],
        out_specs=pl.BlockSpec((EBLK, DPADH), lambda i: (i, 0)),
        out_shape=jax.ShapeDtypeStruct((erows, DPADH), jnp.float32),
    )(ea, xj, w1p, b1r, w2p, b2r)


def _bn_body(p_ref, gamma_ref, beta_ref, out_ref):
    a = jnp.maximum(p_ref[...], 0.0)
    valid = lax.broadcasted_iota(jnp.int32, (NPAD, DPADH), 0) < N_NODES
    a = jnp.where(valid, a, 0.0)
    mean = jnp.sum(a, axis=0, keepdims=True) * (1.0 / N_NODES)
    sq = jnp.where(valid, (a - mean) * (a - mean), 0.0)
    var = jnp.sum(sq, axis=0, keepdims=True) * (1.0 / N_NODES)
    h = (a - mean) * lax.rsqrt(var + 1e-5) * gamma_ref[...] + beta_ref[...]
    out_ref[...] = jnp.where(valid, h, 0.0)


def _bn_call(parts, gamma128, beta128):
    return pl.pallas_call(
        _bn_body,
        grid=(1,),
        in_specs=[
            pl.BlockSpec((NPAD, DPADH), lambda i: (0, 0)),
            pl.BlockSpec((1, DPADH), lambda i: (0, 0)),
            pl.BlockSpec((1, DPADH), lambda i: (0, 0)),
        ],
        out_specs=pl.BlockSpec((NPAD, DPADH), lambda i: (0, 0)),
        out_shape=jax.ShapeDtypeStruct((NPAD, DPADH), jnp.float32),
    )(parts, gamma128, beta128)


def _readout_body(h_ref, batch_ref, wr_ref, br_ref, out_ref, pooled_acc, counts_acc):
    i = pl.program_id(0)

    @pl.when(i == 0)
    def _():
        pooled_acc[...] = jnp.zeros_like(pooled_acc)
        counts_acc[...] = jnp.zeros_like(counts_acc)

    b = batch_ref[0]
    oh = (
        lax.broadcasted_iota(jnp.int32, (NUM_GRAPHS, NBLK), 0) == b
    ).astype(jnp.float32)
    pooled_acc[...] += jnp.dot(
        oh, h_ref[...], preferred_element_type=jnp.float32
    )
    counts_acc[...] += jnp.dot(
        oh, jnp.ones((NBLK, DIM_OUTPUT), jnp.float32),
        preferred_element_type=jnp.float32,
    )

    @pl.when(i == (NPAD // NBLK) - 1)
    def _():
        out_ref[...] = (
            jnp.dot(pooled_acc[...], wr_ref[...], preferred_element_type=jnp.float32)
            + br_ref[...]
        ) / counts_acc[...]


def _readout_call(h, batch3d, wr128, br):
    grid = (NPAD // NBLK,)
    return pl.pallas_call(
        _readout_body,
        grid=grid,
        in_specs=[
            pl.BlockSpec((NBLK, DPADH), lambda i: (i, 0)),
            pl.BlockSpec((1, 1, NBLK), lambda i: (i, 0, 0)),
            pl.BlockSpec((DPADH, DIM_OUTPUT), lambda i: (0, 0)),
            pl.BlockSpec((1, DIM_OUTPUT), lambda i: (0, 0)),
        ],
        out_specs=pl.BlockSpec((NUM_GRAPHS, DIM_OUTPUT), lambda i: (0, 0)),
        out_shape=jax.ShapeDtypeStruct((NUM_GRAPHS, DIM_OUTPUT), jnp.float32),
        scratch_shapes=[
            pltpu.VMEM((NUM_GRAPHS, DPADH), jnp.float32),
            pltpu.VMEM((NUM_GRAPHS, DIM_OUTPUT), jnp.float32),
        ],
    )(h, batch3d, wr128, br)


def _prep_layer(d_in, dpad, w1, b1, w2, b2):
    w1p = jnp.zeros((16, DIM_EDGE_EMBED), jnp.float32).at[:w1.shape[0]].set(w1)
    b1r = b1.reshape(1, DIM_EDGE_EMBED)
    w2p = (
        w2.reshape(DIM_EDGE_EMBED, d_in, DIM_HIDDEN)
        .transpose(1, 2, 0)
        .reshape(d_in, DIM_HIDDEN * DIM_EDGE_EMBED)
    )
    w2p = jnp.zeros((dpad, DIM_HIDDEN * DIM_EDGE_EMBED), jnp.float32).at[:d_in].set(w2p)
    b2r = jnp.zeros((dpad, DPADH), jnp.float32).at[:d_in, :DIM_HIDDEN].set(
        b2.reshape(d_in, DIM_HIDDEN)
    )
    return w1p, b1r, w2p, b2r


def _pad_cols(v, n):
    return jnp.zeros((1, n), jnp.float32).at[0, : v.shape[0]].set(v)


def kernel(x, edge_index, edge_attr, batch,
           l0_w1, l0_b1, l0_w2, l0_b2, l0_gamma, l0_beta,
           l1_w1, l1_b1, l1_w2, l1_b2, l1_gamma, l1_beta,
           w_readout, b_readout):
    src = edge_index[0]
    dst = edge_index[1]
    epad_extra = EPAD - N_EDGES

    x_pad = jnp.zeros((NPAD, 256), jnp.float32).at[:N_NODES, :133].set(x)
    ea_pad = jnp.zeros((EPAD, 16), jnp.float32).at[:N_EDGES, :14].set(edge_attr)
    src2d = jnp.concatenate(
        [src, jnp.zeros((epad_extra,), jnp.int32)]
    ).reshape(EPAD // 128, 128)
    dst2d = jnp.concatenate(
        [dst, jnp.full((epad_extra,), N_NODES, jnp.int32)]
    ).reshape(EPAD // 128, 128)
    batch3d = jnp.concatenate(
        [batch, jnp.full((NPAD - N_NODES,), NUM_GRAPHS, jnp.int32)]
    ).reshape(NPAD // NBLK, 1, NBLK)
    zeros_tbl = jnp.zeros((NPAD, DPADH), jnp.float32)
    ea_a, ea_b = ea_pad[:EPAD // 2], ea_pad[EPAD // 2:]
    src_a, src_b = src2d[: EPAD // 256], src2d[EPAD // 256:]
    dst_a, dst_b = dst2d[: EPAD // 256], dst2d[EPAD // 256:]

    p0 = _prep_layer(133, 256, l0_w1, l0_b1, l0_w2, l0_b2)
    p1 = _prep_layer(32, 128, l1_w1, l1_b1, l1_w2, l1_b2)
    wr128 = jnp.zeros((DPADH, DIM_OUTPUT), jnp.float32).at[:DIM_HIDDEN].set(w_readout)

    ehalf = EPAD // 2

    def layer(table, dgat, dpad, prm, gamma, beta):
        gat = _make_gather(dgat, ehalf)
        sct = _make_scatter(DPADH, ehalf, NPAD)
        xj_a = gat(table, src_a)
        xj_b = gat(table, src_b)
        msg_a = _msg_call(dpad, ehalf, ea_a, xj_a, *prm)
        msg_b = _msg_call(dpad, ehalf, ea_b, xj_b, *prm)
        agg_a = sct(msg_a, dst_a, zeros_tbl)
        agg = sct(msg_b, dst_b, agg_a)
        return _bn_call(agg, _pad_cols(gamma, DPADH), _pad_cols(beta, DPADH))

    h = layer(x_pad, 256, 256, p0, l0_gamma, l0_beta)
    h1 = layer(h, DPADH, 128, p1, l1_gamma, l1_beta)

    return _readout_call(h1, batch3d, wr128, b_readout.reshape(1, DIM_OUTPUT))


# speedup vs baseline: 1.0012x; 1.0012x over previous
import functools

import jax
import jax.numpy as jnp
from jax import lax
from jax.experimental import pallas as pl
from jax.experimental.pallas import tpu as pltpu
from jax.experimental.pallas import tpu_sc as plsc

N_NODES = 10000
N_EDGES = 30000
NUM_GRAPHS = 512
DIM_HIDDEN = 32
DIM_EDGE_EMBED = 128
DIM_OUTPUT = 64

NPAD = 10240
EPAD = 32768
DPADH = 128
EBLK = 1024
NBLK = 1024
NC, NS = 2, 16
NW = NC * NS


@functools.lru_cache(maxsize=None)
def _make_gather(d, erows):
    rows_per_w = erows // NW
    k = rows_per_w // 128
    mesh = plsc.VectorSubcoreMesh(
        core_axis_name="c", subcore_axis_name="s", num_cores=NC, num_subcores=NS
    )

    @functools.partial(
        pl.kernel,
        out_type=jax.ShapeDtypeStruct((erows, d), jnp.float32),
        mesh=mesh,
        scratch_types=[
            pltpu.VMEM((k, 128), jnp.int32),
            pltpu.VMEM((128, d), jnp.float32),
            pltpu.VMEM((128, d), jnp.float32),
            pltpu.VMEM((128, d), jnp.float32),
            pltpu.SemaphoreType.DMA,
            pltpu.SemaphoreType.DMA,
            pltpu.SemaphoreType.DMA,
        ],
    )
    def gather(table_hbm, idx_hbm, out_hbm, idx_v,
               rows_a, rows_b, rows_c, sem_a, sem_b, sem_c):
        cid = lax.axis_index("c")
        sid = lax.axis_index("s")
        wid = sid * NC + cid
        pltpu.sync_copy(idx_hbm.at[pl.ds(wid * k, k)], idx_v)

        bufs = [(rows_a, sem_a), (rows_b, sem_b), (rows_c, sem_c)]
        depth = 3
        copies = [None] * k
        for j in range(min(depth, k)):
            buf, sem = bufs[j % depth]
            copies[j] = pltpu.async_copy(table_hbm.at[idx_v.at[j]], buf, sem)
        for j in range(k):
            copies[j].wait()
            pltpu.sync_copy(
                bufs[j % depth][0],
                out_hbm.at[pl.ds(wid * rows_per_w + j * 128, 128)],
            )
            if j + depth < k:
                buf, sem = bufs[(j + depth) % depth]
                copies[j + depth] = pltpu.async_copy(
                    table_hbm.at[idx_v.at[j + depth]], buf, sem
                )

    return gather


@functools.lru_cache(maxsize=None)
def _make_scatter(d, erows, trows):
    nch = erows // 128
    nmine = nch // NS
    half = trows // 2
    tloc = half + 128
    per_tile = half // NS
    mesh = plsc.VectorSubcoreMesh(
        core_axis_name="c", subcore_axis_name="s", num_cores=NC, num_subcores=NS
    )

    @functools.partial(
        pl.kernel,
        out_type=jax.ShapeDtypeStruct((trows, d), jnp.float32),
        mesh=mesh,
        scratch_types=[
            pltpu.VMEM((nmine, 128), jnp.int32),
            pltpu.VMEM((128, d), jnp.float32),
            pltpu.VMEM((128, d), jnp.float32),
            pltpu.VMEM((per_tile, d), jnp.float32),
            pltpu.VMEM_SHARED((tloc, d), jnp.float32),
            pltpu.SemaphoreType.DMA,
            pltpu.SemaphoreType.DMA,
        ],
    )
    def scatter(msg_hbm, idx_hbm, init_hbm, out_hbm, idx_v, rows_a, rows_b,
                stage_v, acc_sh, sem_a, sem_b):
        cid = lax.axis_index("c")
        sid = lax.axis_index("s")
        lo = cid * half
        base = sid * nmine
        bufs = [(rows_a, sem_a), (rows_b, sem_b)]
        copies = [None] * nmine
        copies[0] = pltpu.async_copy(
            msg_hbm.at[pl.ds(base * 128, 128)], rows_a, sem_a
        )
        pltpu.sync_copy(idx_hbm.at[pl.ds(base, nmine)], idx_v)
        pltpu.sync_copy(init_hbm.at[pl.ds(lo + sid * per_tile, per_tile)], stage_v)
        pltpu.sync_copy(stage_v, acc_sh.at[pl.ds(sid * per_tile, per_tile)])
        for t in range(nmine):
            for q in range(8):
                v = idx_v[t, pl.ds(q * 16, 16)] - lo
                ok = (v >= 0) & (v < half)
                idx_v[t, pl.ds(q * 16, 16)] = jnp.where(ok, v, half)
        plsc.subcore_barrier()

        for t in range(nmine):
            if t + 1 < nmine:
                buf, sem = bufs[(t + 1) % 2]
                copies[t + 1] = pltpu.async_copy(
                    msg_hbm.at[pl.ds((base + t + 1) * 128, 128)], buf, sem
                )
            copies[t].wait()
            pltpu.sync_copy(bufs[t % 2][0], acc_sh.at[idx_v.at[t]], add=True)
        plsc.subcore_barrier()
        pltpu.sync_copy(acc_sh.at[pl.ds(sid * per_tile, per_tile)], stage_v)
        pltpu.sync_copy(
            stage_v, out_hbm.at[pl.ds(lo + sid * per_tile, per_tile)]
        )

    return scatter


def _msg_body(ea_ref, xj_ref, w1_ref, b1_ref, w2_ref, b2r_ref, out_ref):
    e = jnp.maximum(
        jnp.dot(ea_ref[...], w1_ref[...], preferred_element_type=jnp.float32)
        + b1_ref[...],
        0.0,
    )
    t2 = jnp.dot(xj_ref[...], w2_ref[...], preferred_element_type=jnp.float32)
    u = (t2.reshape(EBLK, DIM_HIDDEN, 128) * e[:, None, :]).sum(axis=-1)
    u128 = jnp.concatenate(
        [u, jnp.zeros((EBLK, DPADH - DIM_HIDDEN), jnp.float32)], axis=-1
    )
    out_ref[...] = u128 + jnp.dot(
        xj_ref[...], b2r_ref[...], preferred_element_type=jnp.float32
    )


def _msg_call(dpad, erows, ea, xj, w1p, b1r, w2p, b2r):
    grid = (erows // EBLK,)
    return pl.pallas_call(
        _msg_body,
        grid=grid,
        in_specs=[
            pl.BlockSpec((EBLK, 16), lambda i: (i, 0)),
            pl.BlockSpec((EBLK, dpad), lambda i: (i, 0)),
            pl.BlockSpec((16, 128), lambda i: (0, 0)),
            pl.BlockSpec((1, 128), lambda i: (0, 0)),
            pl.BlockSpec((dpad, DIM_HIDDEN * 128), lambda i: (0, 0)),
            pl.BlockSpec((dpad, DPADH), lambda i: (0, 0)),
        ],
        out_specs=pl.BlockSpec((EBLK, DPADH), lambda i: (i, 0)),
        out_shape=jax.ShapeDtypeStruct((erows, DPADH), jnp.float32),
    )(ea, xj, w1p, b1r, w2p, b2r)


def _bn_body(p_ref, gamma_ref, beta_ref, out_ref):
    a = jnp.maximum(p_ref[...], 0.0)
    valid = lax.broadcasted_iota(jnp.int32, (NPAD, DPADH), 0) < N_NODES
    a = jnp.where(valid, a, 0.0)
    mean = jnp.sum(a, axis=0, keepdims=True) * (1.0 / N_NODES)
    sq = jnp.where(valid, (a - mean) * (a - mean), 0.0)
    var = jnp.sum(sq, axis=0, keepdims=True) * (1.0 / N_NODES)
    h = (a - mean) * lax.rsqrt(var + 1e-5) * gamma_ref[...] + beta_ref[...]
    out_ref[...] = jnp.where(valid, h, 0.0)


def _bn_call(parts, gamma128, beta128):
    return pl.pallas_call(
        _bn_body,
        grid=(1,),
        in_specs=[
            pl.BlockSpec((NPAD, DPADH), lambda i: (0, 0)),
            pl.BlockSpec((1, DPADH), lambda i: (0, 0)),
            pl.BlockSpec((1, DPADH), lambda i: (0, 0)),
        ],
        out_specs=pl.BlockSpec((NPAD, DPADH), lambda i: (0, 0)),
        out_shape=jax.ShapeDtypeStruct((NPAD, DPADH), jnp.float32),
    )(parts, gamma128, beta128)


def _readout_body(h_ref, batch_ref, wr_ref, br_ref, out_ref, pooled_acc, counts_acc):
    i = pl.program_id(0)

    @pl.when(i == 0)
    def _():
        pooled_acc[...] = jnp.zeros_like(pooled_acc)
        counts_acc[...] = jnp.zeros_like(counts_acc)

    b = batch_ref[0]
    oh = (
        lax.broadcasted_iota(jnp.int32, (NUM_GRAPHS, NBLK), 0) == b
    ).astype(jnp.float32)
    pooled_acc[...] += jnp.dot(
        oh, h_ref[...], preferred_element_type=jnp.float32
    )
    counts_acc[...] += jnp.dot(
        oh, jnp.ones((NBLK, DIM_OUTPUT), jnp.float32),
        preferred_element_type=jnp.float32,
    )

    @pl.when(i == (NPAD // NBLK) - 1)
    def _():
        out_ref[...] = (
            jnp.dot(pooled_acc[...], wr_ref[...], preferred_element_type=jnp.float32)
            + br_ref[...]
        ) / counts_acc[...]


def _readout_call(h, batch3d, wr128, br):
    grid = (NPAD // NBLK,)
    return pl.pallas_call(
        _readout_body,
        grid=grid,
        in_specs=[
            pl.BlockSpec((NBLK, DPADH), lambda i: (i, 0)),
            pl.BlockSpec((1, 1, NBLK), lambda i: (i, 0, 0)),
            pl.BlockSpec((DPADH, DIM_OUTPUT), lambda i: (0, 0)),
            pl.BlockSpec((1, DIM_OUTPUT), lambda i: (0, 0)),
        ],
        out_specs=pl.BlockSpec((NUM_GRAPHS, DIM_OUTPUT), lambda i: (0, 0)),
        out_shape=jax.ShapeDtypeStruct((NUM_GRAPHS, DIM_OUTPUT), jnp.float32),
        scratch_shapes=[
            pltpu.VMEM((NUM_GRAPHS, DPADH), jnp.float32),
            pltpu.VMEM((NUM_GRAPHS, DIM_OUTPUT), jnp.float32),
        ],
    )(h, batch3d, wr128, br)


def _prep_layer(d_in, dpad, w1, b1, w2, b2):
    w1p = jnp.zeros((16, DIM_EDGE_EMBED), jnp.float32).at[:w1.shape[0]].set(w1)
    b1r = b1.reshape(1, DIM_EDGE_EMBED)
    w2p = (
        w2.reshape(DIM_EDGE_EMBED, d_in, DIM_HIDDEN)
        .transpose(1, 2, 0)
        .reshape(d_in, DIM_HIDDEN * DIM_EDGE_EMBED)
    )
    w2p = jnp.zeros((dpad, DIM_HIDDEN * DIM_EDGE_EMBED), jnp.float32).at[:d_in].set(w2p)
    b2r = jnp.zeros((dpad, DPADH), jnp.float32).at[:d_in, :DIM_HIDDEN].set(
        b2.reshape(d_in, DIM_HIDDEN)
    )
    return w1p, b1r, w2p, b2r


def _pad_cols(v, n):
    return jnp.zeros((1, n), jnp.float32).at[0, : v.shape[0]].set(v)


def kernel(x, edge_index, edge_attr, batch,
           l0_w1, l0_b1, l0_w2, l0_b2, l0_gamma, l0_beta,
           l1_w1, l1_b1, l1_w2, l1_b2, l1_gamma, l1_beta,
           w_readout, b_readout):
    src = edge_index[0]
    dst = edge_index[1]
    epad_extra = EPAD - N_EDGES

    x_pad = jnp.zeros((NPAD, 256), jnp.float32).at[:N_NODES, :133].set(x)
    ea_pad = jnp.zeros((EPAD, 16), jnp.float32).at[:N_EDGES, :14].set(edge_attr)
    src2d = jnp.concatenate(
        [src, jnp.zeros((epad_extra,), jnp.int32)]
    ).reshape(EPAD // 128, 128)
    dst2d = jnp.concatenate(
        [dst, jnp.full((epad_extra,), N_NODES, jnp.int32)]
    ).reshape(EPAD // 128, 128)
    batch3d = jnp.concatenate(
        [batch, jnp.full((NPAD - N_NODES,), NUM_GRAPHS, jnp.int32)]
    ).reshape(NPAD // NBLK, 1, NBLK)
    zeros_tbl = jnp.zeros((NPAD, DPADH), jnp.float32)
    ea_a, ea_b = ea_pad[:EPAD // 2], ea_pad[EPAD // 2:]
    src_a, src_b = src2d[: EPAD // 256], src2d[EPAD // 256:]
    dst_a, dst_b = dst2d[: EPAD // 256], dst2d[EPAD // 256:]

    p0 = _prep_layer(133, 256, l0_w1, l0_b1, l0_w2, l0_b2)
    p1 = _prep_layer(32, 128, l1_w1, l1_b1, l1_w2, l1_b2)
    wr128 = jnp.zeros((DPADH, DIM_OUTPUT), jnp.float32).at[:DIM_HIDDEN].set(w_readout)

    ehalf = EPAD // 2

    def layer(table, dgat, dpad, prm, gamma, beta):
        gat = _make_gather(dgat, ehalf)
        sct = _make_scatter(DPADH, ehalf, NPAD)
        xj_a = gat(table, src_a)
        xj_b = gat(table, src_b)
        msg_a = _msg_call(dpad, ehalf, ea_a, xj_a, *prm)
        msg_b = _msg_call(dpad, ehalf, ea_b, xj_b, *prm)
        agg_a = sct(msg_a, dst_a, zeros_tbl)
        agg = sct(msg_b, dst_b, agg_a)
        return _bn_call(agg, _pad_cols(gamma, DPADH), _pad_cols(beta, DPADH))

    h = layer(x_pad, 256, 256, p0, l0_gamma, l0_beta)
    h1 = layer(h, DPADH, 128, p1, l1_gamma, l1_beta)

    return _readout_call(h1, batch3d, wr128, b_readout.reshape(1, DIM_OUTPUT))
